# Initial kernel scaffold; baseline (speedup 1.0000x reference)
import jax
import jax.numpy as jnp
from jax.experimental import pallas as pl


def kernel(table0, table1, x):
    raise NotImplementedError("write your pallas kernel here")


# speedup vs baseline: 2.4828x; 2.4828x over previous
import functools

import jax
import jax.numpy as jnp
from jax import lax
from jax.experimental import pallas as pl
from jax.experimental.pallas import tpu as pltpu
from jax.experimental.pallas import tpu_sc as plsc

V0 = 1000000
V1 = 100000
D = 64
NC = 2
NS = 16
LANES = 16
NW = NC * NS

K = 128
P = 64


@functools.lru_cache(maxsize=None)
def _build(N):
    assert N % NW == 0
    chunk = N // NW
    assert chunk % K == 0
    nj = chunk // K
    trash = chunk + P
    m1cap = chunk + P + 16

    mesh = plsc.VectorSubcoreMesh(core_axis_name="c", subcore_axis_name="s")

    @functools.partial(
        pl.kernel,
        out_type=jax.ShapeDtypeStruct((N, D), jnp.float32),
        mesh=mesh,
        compiler_params=pltpu.CompilerParams(
            use_tc_tiling_on_sc=False, needs_layout_passes=False),
        scratch_types=[
            pltpu.VMEM((K,), jnp.int32),
            pltpu.VMEM((K,), jnp.int32),
            pltpu.VMEM((K, D), jnp.float32),
            pltpu.VMEM((m1cap,), jnp.int32),
            pltpu.VMEM((m1cap,), jnp.int32),
            pltpu.VMEM((P,), jnp.int32),
            pltpu.VMEM((P,), jnp.int32),
            pltpu.VMEM((P, D), jnp.float32),
            pltpu.SemaphoreType.DMA,
            pltpu.SemaphoreType.DMA,
        ],
    )
    def emb(t0, t1, xf, out, idxv, l0, r0, list1, pos1, plist, ppos, prows,
            semg, semp):
        wid = lax.axis_index("s") * NC + lax.axis_index("c")
        base = wid * chunk

        def consume_block(o):
            for q in range(P // LANES):
                sl = pl.ds(q * LANES, LANES)
                plist[sl] = list1[pl.ds(o + q * LANES, LANES)]
                ppos[sl] = pos1[pl.ds(o + q * LANES, LANES)]
            pltpu.async_copy(t1.at[plist], prows, semp).wait()
            pltpu.async_copy(prows, out.at[ppos], semp).wait()

        def subchunk(j, carry):
            n1, n1_done = carry
            off = base + j * K
            pltpu.sync_copy(xf.at[pl.ds(off, K)], idxv)

            def vec(v, n1_):
                sl = pl.ds(v * LANES, LANES)
                vi = idxv[sl]
                m1 = vi >= V0
                l0[sl] = jnp.minimum(vi, V0 - 1)
                gpos = (off + v * LANES
                        + lax.broadcasted_iota(jnp.int32, (LANES,), 0))
                i1 = jnp.clip(vi - V0, 0, V1 - 1)
                m1i = m1.astype(jnp.int32)
                excl = plsc.cumsum(m1i) - m1i
                dst = jnp.where(m1, n1_ + excl, jnp.int32(trash))
                plsc.store_scatter(list1, [dst], i1)
                plsc.store_scatter(pos1, [dst], gpos)
                return n1_ + jnp.sum(m1i)

            n1 = lax.fori_loop(0, K // LANES, vec, n1)

            pltpu.async_copy(t0.at[l0], r0, semg).wait()
            pltpu.sync_copy(r0, out.at[pl.ds(off, K)])

            def have_block(nd):
                return nd + P <= n1

            def do_block(nd):
                consume_block(nd)
                return nd + P

            n1_done = lax.while_loop(have_block, do_block, n1_done)
            return (n1, n1_done)

        n1, n1_done = lax.fori_loop(
            0, nj, subchunk, (jnp.int32(0), jnp.int32(0)))

        zero16 = jnp.zeros((LANES,), jnp.int32)
        dup_l = plsc.load_gather(list1, [zero16])
        dup_p = plsc.load_gather(pos1, [zero16])
        for q in range(P // LANES):
            list1[pl.ds(n1 + q * LANES, LANES)] = dup_l
            pos1[pl.ds(n1 + q * LANES, LANES)] = dup_p

        def tail_left(nd):
            return nd < n1

        def tail_block(nd):
            consume_block(nd)
            return nd + P

        lax.while_loop(tail_left, tail_block, n1_done)

    return emb


@jax.jit
def kernel(table0, table1, x):
    B, L = x.shape
    n = B * L
    xf = x.reshape(n)
    out = _build(n)(table0, table1, xf)
    return out.reshape(B, L, D)
